# Initial kernel scaffold; baseline (speedup 1.0000x reference)
import jax
import jax.numpy as jnp
from jax.experimental import pallas as pl


def kernel(x, Win, b_in, Wout, b_out):
    raise NotImplementedError("write your pallas kernel here")


# speedup vs baseline: 1.9875x; 1.9875x over previous
import jax
import jax.numpy as jnp
import numpy as np
from jax.experimental import pallas as pl
from jax.experimental.pallas import tpu as pltpu

_G = 2
_R = 2
_CDIM = 4
_DPG = 512
_LEVELS = 5.0
_HALF_L = (_LEVELS - 1.0) * (1.0 + 1e-3) / 2.0
_HALF_W = 2.0
_BASIS = (1.0, 5.0, 25.0, 125.0)
_TT = 512


def _fsq_kernel(w_ref, b_ref, basis_ref, x_ref, o_ref):
    xb = x_ref[0]
    w = w_ref[...]
    b = b_ref[...]
    z = jax.lax.dot_general(
        w.astype(jnp.bfloat16), xb.astype(jnp.bfloat16), (((1,), (0,)), ((), ())),
        preferred_element_type=jnp.float32,
    ) + b
    r0 = jnp.round(jnp.tanh(z) * _HALF_L)
    resid = z - r0 * (1.0 / _HALF_W)
    r1 = jnp.round(jnp.tanh(resid * 4.0) * _HALF_L)
    basis8 = basis_ref[...]
    w0 = (r0 + _HALF_W) * basis8
    w1 = (r1 + _HALF_W) * basis8
    row = [
        jnp.sum(w0[0:4], axis=0, keepdims=True),
        jnp.sum(w1[0:4], axis=0, keepdims=True),
        jnp.sum(w0[4:8], axis=0, keepdims=True),
        jnp.sum(w1[4:8], axis=0, keepdims=True),
    ]
    o_ref[0] = jnp.concatenate(row, axis=0).astype(jnp.int32)


def kernel(x, Win, b_in, Wout, b_out):
    del Wout, b_out
    B, D, T = x.shape
    w8 = jnp.zeros((_G * _CDIM, D), dtype=jnp.float32)
    w8 = w8.at[0:4, 0:512].set(Win[0]).at[4:8, 512:1024].set(Win[1])
    b8 = jnp.concatenate([b_in[0], b_in[1]]).reshape(_G * _CDIM, 1)
    basis8 = jnp.asarray(_BASIS * _G, dtype=jnp.float32).reshape(_G * _CDIM, 1)
    grid = (B, T // _TT)
    out = pl.pallas_call(
        _fsq_kernel,
        grid=grid,
        in_specs=[
            pl.BlockSpec((_G * _CDIM, D), lambda bi, ti: (0, 0)),
            pl.BlockSpec((_G * _CDIM, 1), lambda bi, ti: (0, 0)),
            pl.BlockSpec((_G * _CDIM, 1), lambda bi, ti: (0, 0)),
            pl.BlockSpec((1, D, _TT), lambda bi, ti: (bi, 0, ti)),
        ],
        out_specs=pl.BlockSpec((1, _G * _R, _TT), lambda bi, ti: (bi, 0, ti)),
        out_shape=jax.ShapeDtypeStruct((B, _G * _R, T), jnp.int32),
    )(w8, b8, basis8, x)
    return out
